# baseline (speedup 1.0000x reference)
import jax
import jax.numpy as jnp
from jax import lax
from jax.experimental import pallas as pl
from jax.experimental.pallas import tpu as pltpu
from jax.experimental.pallas import tpu_sc as plsc

T = 8
N = 10000
NP = 10240
E = 160000
D = 128
HD = 64
H = 16
W1 = HD + H
CH = 128
CPB = 80
CPBP = CPB + 2
EPT = CPB * CH
EP = 16 * EPT
GRP = CH // 16
BN = 400
NBLK = N // BN
RPT = NP // 16
NCHZ = RPT // CH
OPT = N // 16
ECH = OPT // 5


def _bcast_lane(vec, u):
    idx = lax.full((16,), u, jnp.int32)
    dn = lax.GatherDimensionNumbers(offset_dims=(), collapsed_slice_dims=(0,),
                                    start_index_map=(0,))
    return lax.gather(vec, idx[:, None], dn, (1,),
                      mode=lax.GatherScatterMode.PROMISE_IN_BOUNDS)


def _zero_accum(accum, sbuf0, sbuf1, s, width):
    def zrow(i, _):
        for q in range(width // H):
            sbuf0[i, pl.ds(q * H, H)] = jnp.zeros((H,), jnp.float32)
            sbuf1[i, pl.ds(q * H, H)] = jnp.zeros((H,), jnp.float32)
        return 0
    lax.fori_loop(0, CH, zrow, 0)
    zbase = s * RPT
    for k in range(NCHZ):
        pltpu.sync_copy(sbuf0, accum.at[pl.ds(zbase + k * CH, CH)])


def _seg_pipeline(tbl, accum, src_v, dst_v, w_v, bufs, with_wsum, emask):
    def start_gather(ci, b):
        gb, _, gs, _ = bufs[b]
        pltpu.async_copy(tbl.at[src_v.at[ci]], gb, gs)

    def compute(ci, gb, sb):
        @plsc.parallel_loop(0, GRP, step=1)
        def grp(k):
            wrow = w_v[ci * GRP + k]
            for u in range(16):
                e = k * 16 + u
                wb = _bcast_lane(wrow, u)
                for q in range(2):
                    a, b_ = plsc.unpack(
                        gb[e, pl.ds(q * 32, 32)],
                        format=plsc.PackFormat.INTERLEAVED,
                        preferred_element_type=jnp.float32)
                    sb[e, pl.ds(q * 32, H)] = a * wb
                    sb[e, pl.ds(q * 32 + H, H)] = b_ * wb
                if with_wsum:
                    sb[e, pl.ds(HD, H)] = wb * emask

    for b in (0, 1):
        _, sb, _, ss = bufs[b]
        pltpu.async_copy(sb, accum.at[dst_v.at[0]], ss, add=True)
    start_gather(0, 0)
    start_gather(1, 1)

    def outer(g, _):
        for b in (0, 1):
            ci = 2 * g + b
            gb, sb, gs, ss = bufs[b]
            pltpu.make_async_copy(tbl.at[src_v.at[ci]], gb, gs).wait()
            pltpu.make_async_copy(sb, accum.at[dst_v.at[ci]], ss).wait()
            compute(ci, gb, sb)
            start_gather(ci + 2, b)
            pltpu.async_copy(sb, accum.at[dst_v.at[ci]], ss, add=True)
        return 0
    lax.fori_loop(0, CPB // 2, outer, 0)

    for b in (0, 1):
        gb, sb, gs, ss = bufs[b]
        pltpu.make_async_copy(sb, accum.at[dst_v.at[0]], ss).wait()
        pltpu.make_async_copy(tbl.at[src_v.at[0]], gb, gs).wait()


def _make_mega1():
    mesh = plsc.VectorSubcoreMesh(core_axis_name="c", subcore_axis_name="s",
                                  num_cores=2, num_subcores=16)

    def body(u0_h, s0_h, bias_h, src_h, dst_h, w_h,
             emb_h, xaf_h, xab_h, a2_h,
             accum, src_v, dst_v, w_v, bias_v,
             gbuf0, gbuf1, sbuf0, sbuf1, s0v, x0v, xfv,
             gsem0, gsem1, ssem0, ssem1, esem):
        c = lax.axis_index("c")
        s = lax.axis_index("s")
        emask = jnp.where(lax.iota(jnp.int32, 16) == 0, 1.0, 0.0)
        bufs = ((gbuf0, sbuf0, gsem0, ssem0), (gbuf1, sbuf1, gsem1, ssem1))

        pltpu.sync_copy(src_h.at[s], src_v)
        pltpu.sync_copy(dst_h.at[s], dst_v)
        pltpu.sync_copy(w_h.at[s], w_v)
        pltpu.sync_copy(bias_h, bias_v)

        _zero_accum(accum, sbuf0, sbuf1, s, W1)
        plsc.subcore_barrier()
        _seg_pipeline(u0_h.at[c], accum, src_v, dst_v, w_v, bufs, True, emask)
        plsc.subcore_barrier()

        def ew_chunk(k, _):
            base = s * OPT + k * ECH
            pltpu.sync_copy(accum.at[pl.ds(base, ECH)],
                            sbuf0.at[pl.ds(0, ECH)])
            pltpu.sync_copy(s0_h.at[c, pl.ds(base, ECH)],
                            s0v.at[pl.ds(0, ECH)])
            for q in range(4):
                def xrow(r, _):
                    wb = _bcast_lane(sbuf0[r, pl.ds(HD, H)], 0)
                    rv = 1.0 / (wb + 1e-9)
                    x0 = (s0v[r, pl.ds(q * H, H)]
                          + sbuf0[r, pl.ds(q * H, H)] * rv
                          + bias_v[pl.ds(q * H, H)])
                    x0v[r] = x0
                    xfv[r, pl.ds(q * H, H)] = jnp.maximum(x0, 0.0)
                    return 0
                lax.fori_loop(0, ECH, xrow, 0)
                pltpu.sync_copy(x0v.at[pl.ds(0, ECH)],
                                emb_h.at[4 * c + q, pl.ds(base, ECH)])
            def prow(r, _):
                for p in range(2):
                    a = xfv[r, pl.ds(32 * p, H)]
                    b_ = xfv[r, pl.ds(32 * p + H, H)]
                    gbuf0[r, pl.ds(32 * p, 32)] = plsc.pack(
                        a, b_, format=plsc.PackFormat.INTERLEAVED)
                return 0
            lax.fori_loop(0, ECH, prow, 0)
            pltpu.sync_copy(xfv.at[pl.ds(0, ECH)],
                            xaf_h.at[c, pl.ds(base, ECH)])
            pltpu.sync_copy(gbuf0.at[pl.ds(0, ECH)],
                            xab_h.at[c, pl.ds(base, ECH)])
            return 0
        lax.fori_loop(0, 5, ew_chunk, 0)
        plsc.subcore_barrier()

        _zero_accum(accum, sbuf0, sbuf1, s, W1)
        plsc.subcore_barrier()
        _seg_pipeline(xab_h.at[c], accum, src_v, dst_v, w_v, bufs, True,
                      emask)
        plsc.subcore_barrier()
        r = pl.ds(s * OPT, OPT)
        pltpu.sync_copy(accum.at[r], a2_h.at[c, r])

    return pl.kernel(
        body,
        out_type=[
            jax.ShapeDtypeStruct((T, N, H), jnp.float32),
            jax.ShapeDtypeStruct((2, N, HD), jnp.float32),
            jax.ShapeDtypeStruct((2, N, HD), jnp.bfloat16),
            jax.ShapeDtypeStruct((2, N, W1), jnp.float32),
        ],
        mesh=mesh,
        scratch_types=[
            pltpu.VMEM_SHARED((NP, W1), jnp.float32),
            pltpu.VMEM((CPBP, CH), jnp.int32),
            pltpu.VMEM((CPBP, CH), jnp.int32),
            pltpu.VMEM((CPBP * GRP, H), jnp.float32),
            pltpu.VMEM((HD,), jnp.float32),
            pltpu.VMEM((CH, HD), jnp.bfloat16),
            pltpu.VMEM((CH, HD), jnp.bfloat16),
            pltpu.VMEM((CH, W1), jnp.float32),
            pltpu.VMEM((CH, W1), jnp.float32),
            pltpu.VMEM((CH, HD), jnp.float32),
            pltpu.VMEM((CH, H), jnp.float32),
            pltpu.VMEM((CH, HD), jnp.float32),
            pltpu.SemaphoreType.DMA,
            pltpu.SemaphoreType.DMA,
            pltpu.SemaphoreType.DMA,
            pltpu.SemaphoreType.DMA,
            pltpu.SemaphoreType.DMA,
        ],
        compiler_params=pltpu.CompilerParams(use_tc_tiling_on_sc=False,
                                             needs_layout_passes=False),
    )


def _make_seg3():
    mesh = plsc.VectorSubcoreMesh(core_axis_name="c", subcore_axis_name="s",
                                  num_cores=2, num_subcores=16)

    def body(table_h, src_h, dst_h, w_h, out_h, accum, src_v, dst_v, w_v,
             gbuf0, gbuf1, sbuf0, sbuf1, gsem0, gsem1, ssem0, ssem1):
        c = lax.axis_index("c")
        s = lax.axis_index("s")
        emask = jnp.where(lax.iota(jnp.int32, 16) == 0, 1.0, 0.0)
        bufs = ((gbuf0, sbuf0, gsem0, ssem0), (gbuf1, sbuf1, gsem1, ssem1))

        pltpu.sync_copy(src_h.at[s], src_v)
        pltpu.sync_copy(dst_h.at[s], dst_v)
        pltpu.sync_copy(w_h.at[s], w_v)
        _zero_accum(accum, sbuf0, sbuf1, s, HD)
        plsc.subcore_barrier()
        _seg_pipeline(table_h.at[c], accum, src_v, dst_v, w_v, bufs, False,
                      emask)
        plsc.subcore_barrier()
        r = pl.ds(s * OPT, OPT)
        pltpu.sync_copy(accum.at[r], out_h.at[c, r])

    return pl.kernel(
        body,
        out_type=jax.ShapeDtypeStruct((2, N, HD), jnp.float32),
        mesh=mesh,
        scratch_types=[
            pltpu.VMEM_SHARED((NP, HD), jnp.float32),
            pltpu.VMEM((CPBP, CH), jnp.int32),
            pltpu.VMEM((CPBP, CH), jnp.int32),
            pltpu.VMEM((CPBP * GRP, H), jnp.float32),
            pltpu.VMEM((CH, HD), jnp.bfloat16),
            pltpu.VMEM((CH, HD), jnp.bfloat16),
            pltpu.VMEM((CH, HD), jnp.float32),
            pltpu.VMEM((CH, HD), jnp.float32),
            pltpu.SemaphoreType.DMA,
            pltpu.SemaphoreType.DMA,
            pltpu.SemaphoreType.DMA,
            pltpu.SemaphoreType.DMA,
        ],
        compiler_params=pltpu.CompilerParams(use_tc_tiling_on_sc=False,
                                             needs_layout_passes=False),
    )


def _ilv(a, b):
    au = lax.bitcast_convert_type(a.astype(jnp.bfloat16),
                                  jnp.uint16).astype(jnp.uint32)
    bu = lax.bitcast_convert_type(b.astype(jnp.bfloat16),
                                  jnp.uint16).astype(jnp.uint32)
    return au | (bu << 16)


def _to_sc(cols):
    return jnp.concatenate([_ilv(cols[0], cols[1]),
                            _ilv(cols[2], cols[3])], axis=1)


def _k1_body(x_ref, w_ref, s0_ref, u0_ref):
    ss, us = [], []
    for t in range(T):
        r = jnp.dot(x_ref[t], w_ref[...], preferred_element_type=jnp.float32)
        ss.append(r[:, :H])
        us.append(r[:, H:])
    s0_ref[0] = jnp.concatenate(ss[:4], axis=1)
    s0_ref[1] = jnp.concatenate(ss[4:], axis=1)
    u0_ref[0] = _to_sc(us[:4])
    u0_ref[1] = _to_sc(us[4:])


def _gate_inputs(xa_ref, a_ref, rden, t):
    sl = pl.ds((t % 4) * H, H)
    p = t // 4
    return xa_ref[p, :, sl], (a_ref[p, :, sl]) * rden


def _k3_body(xa_ref, a2_ref, ws_ref, wn_ref, b_ref,
             c1_ref, h1_ref, h1b_ref):
    rden = 1.0 / (a2_ref[0, :, HD:HD + 1] + 1e-9)
    h1s = []
    for t in range(T):
        xa_t, na_t = _gate_inputs(xa_ref, a2_ref, rden, t)
        g = (jnp.dot(xa_t, ws_ref[...], preferred_element_type=jnp.float32)
             + jnp.dot(na_t, wn_ref[...], preferred_element_type=jnp.float32)
             + b_ref[...])
        i_, g_, o_ = g[:, :H], g[:, 2 * H:3 * H], g[:, 3 * H:]
        c1_t = jax.nn.sigmoid(i_) * jnp.tanh(g_)
        h1_t = jax.nn.sigmoid(o_) * jnp.tanh(c1_t)
        c1_ref[t] = c1_t
        h1s.append(h1_t)
    h1_ref[0] = jnp.concatenate(h1s[:4], axis=1)
    h1_ref[1] = jnp.concatenate(h1s[4:], axis=1)
    h1b_ref[0] = _to_sc(h1s[:4])
    h1b_ref[1] = _to_sc(h1s[4:])


def _k4_body(xa_ref, a2_ref, a3_ref, c1_ref, h1_ref,
             ws_ref, wn_ref, hs_ref, hn_ref, b_ref, lw_ref, lb_ref,
             c2_ref, out_ref):
    rden = 1.0 / (a2_ref[0, :, HD:HD + 1] + 1e-9)
    for t in range(T):
        xa_t, na_t = _gate_inputs(xa_ref, a2_ref, rden, t)
        h1_t, nh_t = _gate_inputs(h1_ref, a3_ref, rden, t)
        g = (jnp.dot(xa_t, ws_ref[...], preferred_element_type=jnp.float32)
             + jnp.dot(na_t, wn_ref[...], preferred_element_type=jnp.float32)
             + jnp.dot(h1_t, hs_ref[...], preferred_element_type=jnp.float32)
             + jnp.dot(nh_t, hn_ref[...], preferred_element_type=jnp.float32)
             + b_ref[...])
        i_, f_, g_, o_ = (g[:, :H], g[:, H:2 * H], g[:, 2 * H:3 * H],
                          g[:, 3 * H:])
        c2_t = (jax.nn.sigmoid(f_) * c1_ref[t]
                + jax.nn.sigmoid(i_) * jnp.tanh(g_))
        c2_ref[t] = c2_t
        if t >= T - 4:
            h2_t = jax.nn.sigmoid(o_) * jnp.tanh(c2_t)
            out_ref[t - (T - 4)] = (jnp.dot(h2_t, lw_ref[...],
                                            preferred_element_type=jnp.float32)
                                    + lb_ref[...])


def _full(shape):
    return pl.BlockSpec(shape, lambda i: tuple(0 for _ in shape))


def _a_block(width):
    return pl.BlockSpec((2, BN, width), lambda i: (0, i, 0))


def _t_block(nt, width):
    return pl.BlockSpec((nt, BN, width), lambda i: (0, i, 0))


_3D = jax.ShapeDtypeStruct((T, N, H), jnp.float32)
_SPLIT = jax.ShapeDtypeStruct((2, N, HD), jnp.float32)
_SPLITB = jax.ShapeDtypeStruct((2, N, HD // 2), jnp.uint32)


def _as_bf16(tbl_u32):
    return lax.bitcast_convert_type(tbl_u32, jnp.bfloat16).reshape(2, N, HD)


_k1 = pl.pallas_call(
    _k1_body,
    grid=(NBLK,),
    in_specs=[_t_block(T, D), _full((D, 2 * H))],
    out_specs=[_a_block(HD), _a_block(HD // 2)],
    out_shape=[_SPLIT, _SPLITB],
)

_k3 = pl.pallas_call(
    _k3_body,
    grid=(NBLK,),
    in_specs=[_a_block(HD), _a_block(W1),
              _full((H, 4 * H)), _full((H, 4 * H)), _full((1, 4 * H))],
    out_specs=[_t_block(T, H), _a_block(HD), _a_block(HD // 2)],
    out_shape=[_3D, _SPLIT, _SPLITB],
)

_k4 = pl.pallas_call(
    _k4_body,
    grid=(NBLK,),
    in_specs=[_a_block(HD), _a_block(W1), _a_block(HD),
              _t_block(T, H), _a_block(HD),
              _full((H, 4 * H)), _full((H, 4 * H)),
              _full((H, 4 * H)), _full((H, 4 * H)), _full((1, 4 * H)),
              _full((H, 1)), _full((1, 1))],
    out_specs=[_t_block(T, H), _t_block(4, 1)],
    out_shape=[_3D, jax.ShapeDtypeStruct((4, N, 1), jnp.float32)],
)

_mega1 = _make_mega1()
_seg3 = _make_seg3()


def kernel(x, edge_index, edge_attr, gs_Ws, gs_Wn, gs_b,
           l1x_Ws, l1x_Wn, l1x_b, l1h_Ws, l1h_Wn, l1h_b,
           l2x_Ws, l2x_Wn, l2x_b, l2h_Ws, l2h_Wn, l2h_b,
           lin_W, lin_b):
    pad = EP - E
    zc = jnp.zeros((16, 2, CH), jnp.int32)
    src = jnp.concatenate([edge_index[0].astype(jnp.int32),
                           jnp.zeros((pad,), jnp.int32)])
    dst = jnp.concatenate([edge_index[1].astype(jnp.int32),
                           jnp.zeros((pad,), jnp.int32)])
    wp = jnp.concatenate([edge_attr, jnp.zeros((pad,), jnp.float32)])
    src = jnp.concatenate([src.reshape(16, CPB, CH), zc], axis=1)
    dst = jnp.concatenate([dst.reshape(16, CPB, CH), zc], axis=1)
    wg = jnp.concatenate([wp.reshape(16, CPB * GRP, H),
                          jnp.zeros((16, 2 * GRP, H), jnp.float32)], axis=1)
    wcat = jnp.concatenate([gs_Ws, gs_Wn], axis=1)

    s0p, u0 = _k1(x, wcat)
    emb, xaf, _, a2 = _mega1(_as_bf16(u0), s0p, jnp.tile(gs_b, 4),
                             src, dst, wg)

    b1 = (l1x_b + l1h_b).reshape(1, 4 * H)
    c1, h12, h1b = _k3(xaf, a2, l1x_Ws, l1x_Wn, b1)

    a3 = _seg3(_as_bf16(h1b), src, dst, wg)
    b2 = (l2x_b + l2h_b).reshape(1, 4 * H)
    c2, out = _k4(xaf, a2, a3, c1, h12,
                  l2x_Ws, l2x_Wn, l2h_Ws, l2h_Wn, b2,
                  lin_W, lin_b.reshape(1, 1))
    return (out, c2, emb)


# speedup vs baseline: 1.0986x; 1.0986x over previous
import jax
import jax.numpy as jnp
from jax import lax
from jax.experimental import pallas as pl
from jax.experimental.pallas import tpu as pltpu
from jax.experimental.pallas import tpu_sc as plsc

T = 8
N = 10000
NP = 10240
E = 160000
D = 128
HD = 64
H = 16
NTILES = 32
CH = 128
CPB = 80
EPT = CPB * CH
EP = (NTILES // 2) * EPT
GRP = CH // 16
BN = 400
NBLK = N // BN
RPT = NP // 16
NCHZ = RPT // CH
OPT = N // 16


def _bcast_lane(vec, u):
    idx = lax.full((16,), u, jnp.int32)
    dn = lax.GatherDimensionNumbers(offset_dims=(), collapsed_slice_dims=(0,),
                                    start_index_map=(0,))
    return lax.gather(vec, idx[:, None], dn, (1,),
                      mode=lax.GatherScatterMode.PROMISE_IN_BOUNDS)


def _make_seg_pass(with_wsum):
    width = HD + H if with_wsum else HD
    mesh = plsc.VectorSubcoreMesh(core_axis_name="c", subcore_axis_name="s",
                                  num_cores=2, num_subcores=16)

    def body(table_h, src_h, dst_h, w_h, out_h, accum, src_v, dst_v, w_v,
             gbuf0, gbuf1, sbuf0, sbuf1, gsem0, gsem1, ssem0, ssem1):
        c = lax.axis_index("c")
        s = lax.axis_index("s")
        tbl = table_h.at[c]
        emask = jnp.where(lax.iota(jnp.int32, 16) == 0, 1.0, 0.0)

        pltpu.sync_copy(src_h.at[s], src_v)
        pltpu.sync_copy(dst_h.at[s], dst_v)
        pltpu.sync_copy(w_h.at[s], w_v)

        def zrow(i, _):
            for q in range(width // H):
                sbuf0[i, pl.ds(q * H, H)] = jnp.zeros((H,), jnp.float32)
            return 0
        lax.fori_loop(0, CH, zrow, 0)
        zbase = s * RPT
        for k in range(NCHZ):
            pltpu.sync_copy(sbuf0, accum.at[pl.ds(zbase + k * CH, CH)])
        plsc.subcore_barrier()

        bufs = ((gbuf0, sbuf0, gsem0, ssem0), (gbuf1, sbuf1, gsem1, ssem1))

        def start_gather(ci, b):
            gb, _, gs, _ = bufs[b]
            pltpu.async_copy(tbl.at[src_v.at[ci]], gb, gs)

        def compute(ci, gb, sb):
            @plsc.parallel_loop(0, GRP, step=1)
            def grp(k):
                wrow = w_v[ci * GRP + k]
                for u in range(16):
                    e = k * 16 + u
                    wb = _bcast_lane(wrow, u)
                    for q in range(2):
                        a, b_ = plsc.unpack(
                            gb[e, pl.ds(q * 32, 32)],
                            format=plsc.PackFormat.INTERLEAVED,
                            preferred_element_type=jnp.float32)
                        sb[e, pl.ds(q * 32, H)] = a * wb
                        sb[e, pl.ds(q * 32 + H, H)] = b_ * wb
                    if with_wsum:
                        sb[e, pl.ds(HD, H)] = wb * emask

        def step(ci, b, swait, gnext, sync_scatter=False):
            gb, sb, gs, ss = bufs[b]
            pltpu.make_async_copy(tbl.at[src_v.at[ci]], gb, gs).wait()
            if swait:
                pltpu.make_async_copy(sb, accum.at[dst_v.at[ci]], ss).wait()
            compute(ci, gb, sb)
            if gnext:
                start_gather(ci + 2, b)
            if sync_scatter:
                pltpu.sync_copy(sb, accum.at[dst_v.at[ci]], add=True)
            else:
                pltpu.async_copy(sb, accum.at[dst_v.at[ci]], ss, add=True)

        start_gather(0, 0)
        start_gather(1, 1)
        step(0, 0, swait=False, gnext=True)
        step(1, 1, swait=False, gnext=True)

        def outer(g, _):
            ci = 2 * g
            step(ci, 0, swait=True, gnext=True)
            step(ci + 1, 1, swait=True, gnext=True)
            return 0
        lax.fori_loop(1, CPB // 2 - 1, outer, 0)
        step(CPB - 2, 0, swait=True, gnext=False, sync_scatter=True)
        step(CPB - 1, 1, swait=True, gnext=False, sync_scatter=True)

        plsc.subcore_barrier()
        r = pl.ds(s * OPT, OPT)
        pltpu.sync_copy(accum.at[r], out_h.at[c, r])

    return pl.kernel(
        body,
        out_type=jax.ShapeDtypeStruct((2, N, width), jnp.float32),
        mesh=mesh,
        scratch_types=[
            pltpu.VMEM_SHARED((NP, width), jnp.float32),
            pltpu.VMEM((CPB, CH), jnp.int32),
            pltpu.VMEM((CPB, CH), jnp.int32),
            pltpu.VMEM((CPB * GRP, H), jnp.float32),
            pltpu.VMEM((CH, HD), jnp.bfloat16),
            pltpu.VMEM((CH, HD), jnp.bfloat16),
            pltpu.VMEM((CH, width), jnp.float32),
            pltpu.VMEM((CH, width), jnp.float32),
            pltpu.SemaphoreType.DMA,
            pltpu.SemaphoreType.DMA,
            pltpu.SemaphoreType.DMA,
            pltpu.SemaphoreType.DMA,
        ],
        compiler_params=pltpu.CompilerParams(use_tc_tiling_on_sc=False,
                                             needs_layout_passes=False),
    )


def _ilv(a, b):
    au = lax.bitcast_convert_type(a.astype(jnp.bfloat16),
                                  jnp.uint16).astype(jnp.uint32)
    bu = lax.bitcast_convert_type(b.astype(jnp.bfloat16),
                                  jnp.uint16).astype(jnp.uint32)
    return au | (bu << 16)


def _to_sc(cols):
    return jnp.concatenate([_ilv(cols[0], cols[1]),
                            _ilv(cols[2], cols[3])], axis=1)


def _k1_body(x_ref, w_ref, s0_ref, u0_ref):
    ss, us = [], []
    for t in range(T):
        r = jnp.dot(x_ref[t], w_ref[...], preferred_element_type=jnp.float32)
        ss.append(r[:, :H])
        us.append(r[:, H:])
    s0_ref[...] = jnp.concatenate(ss, axis=1)
    u0_ref[0] = _to_sc(us[:4])
    u0_ref[1] = _to_sc(us[4:])


def _k2_body(s0_ref, a1_ref, b_ref, emb_ref, xa_ref, xab_ref, rden_ref):
    agg = jnp.concatenate([a1_ref[0, :, :HD], a1_ref[1, :, :HD]], axis=1)
    wsum = a1_ref[0, :, HD:HD + 1]
    rden = 1.0 / (wsum + 1e-9)
    x0 = s0_ref[...] + agg * rden + b_ref[...]
    for t in range(T):
        emb_ref[t] = x0[:, t * H:(t + 1) * H]
    xa = jnp.maximum(x0, 0.0)
    xa_ref[0] = xa[:, :HD]
    xa_ref[1] = xa[:, HD:]
    xac = [xa[:, t * H:(t + 1) * H] for t in range(T)]
    xab_ref[0] = _to_sc(xac[:4])
    xab_ref[1] = _to_sc(xac[4:])
    rden_ref[...] = rden


def _gate_inputs(xa_ref, a_ref, rden, t):
    sl = pl.ds((t % 4) * H, H)
    p = t // 4
    return xa_ref[p, :, sl], (a_ref[p, :, sl]) * rden


def _k3_body(xa_ref, a2_ref, rden_ref, ws_ref, wn_ref, b_ref,
             c1_ref, h1_ref, h1b_ref):
    rden = rden_ref[...]
    h1s = []
    for t in range(T):
        xa_t, na_t = _gate_inputs(xa_ref, a2_ref, rden, t)
        g = (jnp.dot(xa_t, ws_ref[...], preferred_element_type=jnp.float32)
             + jnp.dot(na_t, wn_ref[...], preferred_element_type=jnp.float32)
             + b_ref[...])
        i_, g_, o_ = g[:, :H], g[:, 2 * H:3 * H], g[:, 3 * H:]
        c1_t = jax.nn.sigmoid(i_) * jnp.tanh(g_)
        h1_t = jax.nn.sigmoid(o_) * jnp.tanh(c1_t)
        c1_ref[t] = c1_t
        h1s.append(h1_t)
    h1_ref[0] = jnp.concatenate(h1s[:4], axis=1)
    h1_ref[1] = jnp.concatenate(h1s[4:], axis=1)
    h1b_ref[0] = _to_sc(h1s[:4])
    h1b_ref[1] = _to_sc(h1s[4:])


def _k4_body(xa_ref, a2_ref, a3_ref, rden_ref, c1_ref, h1_ref,
             ws_ref, wn_ref, hs_ref, hn_ref, b_ref, lw_ref, lb_ref,
             c2_ref, out_ref):
    rden = rden_ref[...]
    for t in range(T):
        xa_t, na_t = _gate_inputs(xa_ref, a2_ref, rden, t)
        h1_t, nh_t = _gate_inputs(h1_ref, a3_ref, rden, t)
        g = (jnp.dot(xa_t, ws_ref[...], preferred_element_type=jnp.float32)
             + jnp.dot(na_t, wn_ref[...], preferred_element_type=jnp.float32)
             + jnp.dot(h1_t, hs_ref[...], preferred_element_type=jnp.float32)
             + jnp.dot(nh_t, hn_ref[...], preferred_element_type=jnp.float32)
             + b_ref[...])
        i_, f_, g_, o_ = (g[:, :H], g[:, H:2 * H], g[:, 2 * H:3 * H],
                          g[:, 3 * H:])
        c2_t = (jax.nn.sigmoid(f_) * c1_ref[t]
                + jax.nn.sigmoid(i_) * jnp.tanh(g_))
        c2_ref[t] = c2_t
        if t >= T - 4:
            h2_t = jax.nn.sigmoid(o_) * jnp.tanh(c2_t)
            out_ref[t - (T - 4)] = (jnp.dot(h2_t, lw_ref[...],
                                            preferred_element_type=jnp.float32)
                                    + lb_ref[...])


def _full(shape):
    return pl.BlockSpec(shape, lambda i: tuple(0 for _ in shape))


def _nblock(width):
    return pl.BlockSpec((BN, width), lambda i: (i, 0))


def _a_block(width):
    return pl.BlockSpec((2, BN, width), lambda i: (0, i, 0))


def _t_block(nt, width):
    return pl.BlockSpec((nt, BN, width), lambda i: (0, i, 0))


_DH = jax.ShapeDtypeStruct((N, D), jnp.float32)
_3D = jax.ShapeDtypeStruct((T, N, H), jnp.float32)
_SPLIT = jax.ShapeDtypeStruct((2, N, HD), jnp.float32)
_SPLITB = jax.ShapeDtypeStruct((2, N, HD // 2), jnp.uint32)


def _as_bf16(tbl_u32):
    return lax.bitcast_convert_type(tbl_u32, jnp.bfloat16).reshape(2, N, HD)

_k1 = pl.pallas_call(
    _k1_body,
    grid=(NBLK,),
    in_specs=[_t_block(T, D), _full((D, 2 * H))],
    out_specs=[_nblock(D), _a_block(HD // 2)],
    out_shape=[_DH, _SPLITB],
)

_k2 = pl.pallas_call(
    _k2_body,
    grid=(NBLK,),
    in_specs=[_nblock(D), _a_block(HD + H), _full((1, D))],
    out_specs=[_t_block(T, H), _a_block(HD), _a_block(HD // 2), _nblock(1)],
    out_shape=[_3D, _SPLIT, _SPLITB, jax.ShapeDtypeStruct((N, 1), jnp.float32)],
)

_k3 = pl.pallas_call(
    _k3_body,
    grid=(NBLK,),
    in_specs=[_a_block(HD), _a_block(HD), _nblock(1),
              _full((H, 4 * H)), _full((H, 4 * H)), _full((1, 4 * H))],
    out_specs=[_t_block(T, H), _a_block(HD), _a_block(HD // 2)],
    out_shape=[_3D, _SPLIT, _SPLITB],
)

_k4 = pl.pallas_call(
    _k4_body,
    grid=(NBLK,),
    in_specs=[_a_block(HD), _a_block(HD), _a_block(HD), _nblock(1),
              _t_block(T, H), _a_block(HD),
              _full((H, 4 * H)), _full((H, 4 * H)),
              _full((H, 4 * H)), _full((H, 4 * H)), _full((1, 4 * H)),
              _full((H, 1)), _full((1, 1))],
    out_specs=[_t_block(T, H), _t_block(4, 1)],
    out_shape=[_3D, jax.ShapeDtypeStruct((4, N, 1), jnp.float32)],
)

_seg_pass_w = _make_seg_pass(True)
_seg_pass = _make_seg_pass(False)


def kernel(x, edge_index, edge_attr, gs_Ws, gs_Wn, gs_b,
           l1x_Ws, l1x_Wn, l1x_b, l1h_Ws, l1h_Wn, l1h_b,
           l2x_Ws, l2x_Wn, l2x_b, l2h_Ws, l2h_Wn, l2h_b,
           lin_W, lin_b):
    pad = EP - E
    npt = NTILES // 2
    src = jnp.concatenate([edge_index[0].astype(jnp.int32),
                           jnp.zeros((pad,), jnp.int32)]).reshape(npt, CPB, CH)
    dst = jnp.concatenate([edge_index[1].astype(jnp.int32),
                           jnp.zeros((pad,), jnp.int32)]).reshape(npt, CPB, CH)
    wp = jnp.concatenate([edge_attr, jnp.zeros((pad,), jnp.float32)])
    wg = wp.reshape(npt, CPB * GRP, H)
    wcat = jnp.concatenate([gs_Ws, gs_Wn], axis=1)

    s0T, u0 = _k1(x, wcat)
    a1 = _seg_pass_w(_as_bf16(u0), src, dst, wg)
    emb, xa2, xab, rden = _k2(s0T, a1, jnp.tile(gs_b, T).reshape(1, D))

    a2 = _seg_pass(_as_bf16(xab), src, dst, wg)
    b1 = (l1x_b + l1h_b).reshape(1, 4 * H)
    c1, h12, h1b = _k3(xa2, a2, rden, l1x_Ws, l1x_Wn, b1)

    a3 = _seg_pass(_as_bf16(h1b), src, dst, wg)
    b2 = (l2x_b + l2h_b).reshape(1, 4 * H)
    c2, out = _k4(xa2, a2, a3, rden, c1, h12,
                  l2x_Ws, l2x_Wn, l2h_Ws, l2h_Wn, b2,
                  lin_W, lin_b.reshape(1, 1))
    return (out, c2, emb)


# speedup vs baseline: 1.1670x; 1.0623x over previous
import jax
import jax.numpy as jnp
from jax import lax
from jax.experimental import pallas as pl
from jax.experimental.pallas import tpu as pltpu
from jax.experimental.pallas import tpu_sc as plsc

T = 8
N = 10000
NP = 10240
E = 160000
D = 128
HD = 64
H = 16
NTILES = 32
CH = 128
CPB = 80
EPT = CPB * CH
EP = (NTILES // 2) * EPT
GRP = CH // 16
BN = 1000
NBLK = N // BN
RPT = NP // 16
NCHZ = RPT // CH
OPT = N // 16


def _bcast_lane(vec, u):
    idx = lax.full((16,), u, jnp.int32)
    dn = lax.GatherDimensionNumbers(offset_dims=(), collapsed_slice_dims=(0,),
                                    start_index_map=(0,))
    return lax.gather(vec, idx[:, None], dn, (1,),
                      mode=lax.GatherScatterMode.PROMISE_IN_BOUNDS)


def _make_seg_pass(with_wsum):
    width = HD + H if with_wsum else HD
    mesh = plsc.VectorSubcoreMesh(core_axis_name="c", subcore_axis_name="s",
                                  num_cores=2, num_subcores=16)

    def body(table_h, src_h, dst_h, w_h, out_h, accum, src_v, dst_v, w_v,
             gbuf0, gbuf1, sbuf0, sbuf1, gsem0, gsem1, ssem0, ssem1):
        c = lax.axis_index("c")
        s = lax.axis_index("s")
        tbl = table_h.at[c]
        emask = jnp.where(lax.iota(jnp.int32, 16) == 0, 1.0, 0.0)

        pltpu.sync_copy(src_h.at[s], src_v)
        pltpu.sync_copy(dst_h.at[s], dst_v)
        pltpu.sync_copy(w_h.at[s], w_v)

        def zrow(i, _):
            for q in range(width // H):
                sbuf0[i, pl.ds(q * H, H)] = jnp.zeros((H,), jnp.float32)
            return 0
        lax.fori_loop(0, CH, zrow, 0)
        zbase = s * RPT
        for k in range(NCHZ):
            pltpu.sync_copy(sbuf0, accum.at[pl.ds(zbase + k * CH, CH)])
        plsc.subcore_barrier()

        bufs = ((gbuf0, sbuf0, gsem0, ssem0), (gbuf1, sbuf1, gsem1, ssem1))

        def start_gather(ci, b):
            gb, _, gs, _ = bufs[b]
            pltpu.async_copy(tbl.at[src_v.at[ci]], gb, gs)

        def compute(ci, gb, sb):
            @plsc.parallel_loop(0, GRP, step=1)
            def grp(k):
                wrow = w_v[ci * GRP + k]
                for u in range(16):
                    e = k * 16 + u
                    wb = _bcast_lane(wrow, u)
                    for q in range(2):
                        a, b_ = plsc.unpack(
                            gb[e, pl.ds(q * 32, 32)],
                            format=plsc.PackFormat.INTERLEAVED,
                            preferred_element_type=jnp.float32)
                        sb[e, pl.ds(q * 32, H)] = a * wb
                        sb[e, pl.ds(q * 32 + H, H)] = b_ * wb
                    if with_wsum:
                        sb[e, pl.ds(HD, H)] = wb * emask

        def step(ci, b, swait, gnext, sync_scatter=False):
            gb, sb, gs, ss = bufs[b]
            pltpu.make_async_copy(tbl.at[src_v.at[ci]], gb, gs).wait()
            if swait:
                pltpu.make_async_copy(sb, accum.at[dst_v.at[ci]], ss).wait()
            compute(ci, gb, sb)
            if gnext:
                start_gather(ci + 2, b)
            if sync_scatter:
                pltpu.sync_copy(sb, accum.at[dst_v.at[ci]], add=True)
            else:
                pltpu.async_copy(sb, accum.at[dst_v.at[ci]], ss, add=True)

        start_gather(0, 0)
        start_gather(1, 1)
        step(0, 0, swait=False, gnext=True)
        step(1, 1, swait=False, gnext=True)

        def outer(g, _):
            ci = 2 * g
            step(ci, 0, swait=True, gnext=True)
            step(ci + 1, 1, swait=True, gnext=True)
            return 0
        lax.fori_loop(1, CPB // 2 - 1, outer, 0)
        step(CPB - 2, 0, swait=True, gnext=False, sync_scatter=True)
        step(CPB - 1, 1, swait=True, gnext=False, sync_scatter=True)

        plsc.subcore_barrier()
        r = pl.ds(s * OPT, OPT)
        pltpu.sync_copy(accum.at[r], out_h.at[c, r])

    return pl.kernel(
        body,
        out_type=jax.ShapeDtypeStruct((2, N, width), jnp.float32),
        mesh=mesh,
        scratch_types=[
            pltpu.VMEM_SHARED((NP, width), jnp.float32),
            pltpu.VMEM((CPB, CH), jnp.int32),
            pltpu.VMEM((CPB, CH), jnp.int32),
            pltpu.VMEM((CPB * GRP, H), jnp.float32),
            pltpu.VMEM((CH, HD), jnp.bfloat16),
            pltpu.VMEM((CH, HD), jnp.bfloat16),
            pltpu.VMEM((CH, width), jnp.float32),
            pltpu.VMEM((CH, width), jnp.float32),
            pltpu.SemaphoreType.DMA,
            pltpu.SemaphoreType.DMA,
            pltpu.SemaphoreType.DMA,
            pltpu.SemaphoreType.DMA,
        ],
        compiler_params=pltpu.CompilerParams(use_tc_tiling_on_sc=False,
                                             needs_layout_passes=False),
    )


def _ilv(a, b):
    au = lax.bitcast_convert_type(a.astype(jnp.bfloat16),
                                  jnp.uint16).astype(jnp.uint32)
    bu = lax.bitcast_convert_type(b.astype(jnp.bfloat16),
                                  jnp.uint16).astype(jnp.uint32)
    return au | (bu << 16)


def _to_sc(cols):
    return jnp.concatenate([_ilv(cols[0], cols[1]),
                            _ilv(cols[2], cols[3])], axis=1)


def _k1_body(x_ref, w_ref, s0_ref, u0_ref):
    ss, us = [], []
    for t in range(T):
        r = jnp.dot(x_ref[t], w_ref[...], preferred_element_type=jnp.float32)
        ss.append(r[:, :H])
        us.append(r[:, H:])
    s0_ref[...] = jnp.concatenate(ss, axis=1)
    u0_ref[0] = _to_sc(us[:4])
    u0_ref[1] = _to_sc(us[4:])


def _k2_body(s0_ref, a1_ref, b_ref, emb_ref, xa_ref, xab_ref, rden_ref):
    agg = jnp.concatenate([a1_ref[0, :, :HD], a1_ref[1, :, :HD]], axis=1)
    wsum = a1_ref[0, :, HD:HD + 1]
    rden = 1.0 / (wsum + 1e-9)
    x0 = s0_ref[...] + agg * rden + b_ref[...]
    for t in range(T):
        emb_ref[t] = x0[:, t * H:(t + 1) * H]
    xa = jnp.maximum(x0, 0.0)
    xa_ref[0] = xa[:, :HD]
    xa_ref[1] = xa[:, HD:]
    xac = [xa[:, t * H:(t + 1) * H] for t in range(T)]
    xab_ref[0] = _to_sc(xac[:4])
    xab_ref[1] = _to_sc(xac[4:])
    rden_ref[...] = rden


def _gate_inputs(xa_ref, a_ref, rden, t):
    sl = pl.ds((t % 4) * H, H)
    p = t // 4
    return xa_ref[p, :, sl], (a_ref[p, :, sl]) * rden


def _k3_body(xa_ref, a2_ref, rden_ref, ws_ref, wn_ref, b_ref,
             c1_ref, h1_ref, h1b_ref):
    rden = rden_ref[...]
    h1s = []
    for t in range(T):
        xa_t, na_t = _gate_inputs(xa_ref, a2_ref, rden, t)
        g = (jnp.dot(xa_t, ws_ref[...], preferred_element_type=jnp.float32)
             + jnp.dot(na_t, wn_ref[...], preferred_element_type=jnp.float32)
             + b_ref[...])
        i_, g_, o_ = g[:, :H], g[:, 2 * H:3 * H], g[:, 3 * H:]
        c1_t = jax.nn.sigmoid(i_) * jnp.tanh(g_)
        h1_t = jax.nn.sigmoid(o_) * jnp.tanh(c1_t)
        c1_ref[t] = c1_t
        h1s.append(h1_t)
    h1_ref[0] = jnp.concatenate(h1s[:4], axis=1)
    h1_ref[1] = jnp.concatenate(h1s[4:], axis=1)
    h1b_ref[0] = _to_sc(h1s[:4])
    h1b_ref[1] = _to_sc(h1s[4:])


def _k4_body(xa_ref, a2_ref, a3_ref, rden_ref, c1_ref, h1_ref,
             ws_ref, wn_ref, hs_ref, hn_ref, b_ref, lw_ref, lb_ref,
             c2_ref, out_ref):
    rden = rden_ref[...]
    for t in range(T):
        xa_t, na_t = _gate_inputs(xa_ref, a2_ref, rden, t)
        h1_t, nh_t = _gate_inputs(h1_ref, a3_ref, rden, t)
        g = (jnp.dot(xa_t, ws_ref[...], preferred_element_type=jnp.float32)
             + jnp.dot(na_t, wn_ref[...], preferred_element_type=jnp.float32)
             + jnp.dot(h1_t, hs_ref[...], preferred_element_type=jnp.float32)
             + jnp.dot(nh_t, hn_ref[...], preferred_element_type=jnp.float32)
             + b_ref[...])
        i_, f_, g_, o_ = (g[:, :H], g[:, H:2 * H], g[:, 2 * H:3 * H],
                          g[:, 3 * H:])
        c2_t = (jax.nn.sigmoid(f_) * c1_ref[t]
                + jax.nn.sigmoid(i_) * jnp.tanh(g_))
        c2_ref[t] = c2_t
        if t >= T - 4:
            h2_t = jax.nn.sigmoid(o_) * jnp.tanh(c2_t)
            out_ref[t - (T - 4)] = (jnp.dot(h2_t, lw_ref[...],
                                            preferred_element_type=jnp.float32)
                                    + lb_ref[...])


def _full(shape):
    return pl.BlockSpec(shape, lambda i: tuple(0 for _ in shape))


def _nblock(width):
    return pl.BlockSpec((BN, width), lambda i: (i, 0))


def _a_block(width):
    return pl.BlockSpec((2, BN, width), lambda i: (0, i, 0))


def _t_block(nt, width):
    return pl.BlockSpec((nt, BN, width), lambda i: (0, i, 0))


_DH = jax.ShapeDtypeStruct((N, D), jnp.float32)
_3D = jax.ShapeDtypeStruct((T, N, H), jnp.float32)
_SPLIT = jax.ShapeDtypeStruct((2, N, HD), jnp.float32)
_SPLITB = jax.ShapeDtypeStruct((2, N, HD // 2), jnp.uint32)


def _as_bf16(tbl_u32):
    return lax.bitcast_convert_type(tbl_u32, jnp.bfloat16).reshape(2, N, HD)

_k1 = pl.pallas_call(
    _k1_body,
    grid=(NBLK,),
    in_specs=[_t_block(T, D), _full((D, 2 * H))],
    out_specs=[_nblock(D), _a_block(HD // 2)],
    out_shape=[_DH, _SPLITB],
)

_k2 = pl.pallas_call(
    _k2_body,
    grid=(NBLK,),
    in_specs=[_nblock(D), _a_block(HD + H), _full((1, D))],
    out_specs=[_t_block(T, H), _a_block(HD), _a_block(HD // 2), _nblock(1)],
    out_shape=[_3D, _SPLIT, _SPLITB, jax.ShapeDtypeStruct((N, 1), jnp.float32)],
)

_k3 = pl.pallas_call(
    _k3_body,
    grid=(NBLK,),
    in_specs=[_a_block(HD), _a_block(HD), _nblock(1),
              _full((H, 4 * H)), _full((H, 4 * H)), _full((1, 4 * H))],
    out_specs=[_t_block(T, H), _a_block(HD), _a_block(HD // 2)],
    out_shape=[_3D, _SPLIT, _SPLITB],
)

_k4 = pl.pallas_call(
    _k4_body,
    grid=(NBLK,),
    in_specs=[_a_block(HD), _a_block(HD), _a_block(HD), _nblock(1),
              _t_block(T, H), _a_block(HD),
              _full((H, 4 * H)), _full((H, 4 * H)),
              _full((H, 4 * H)), _full((H, 4 * H)), _full((1, 4 * H)),
              _full((H, 1)), _full((1, 1))],
    out_specs=[_t_block(T, H), _t_block(4, 1)],
    out_shape=[_3D, jax.ShapeDtypeStruct((4, N, 1), jnp.float32)],
)

_seg_pass_w = _make_seg_pass(True)
_seg_pass = _make_seg_pass(False)


def kernel(x, edge_index, edge_attr, gs_Ws, gs_Wn, gs_b,
           l1x_Ws, l1x_Wn, l1x_b, l1h_Ws, l1h_Wn, l1h_b,
           l2x_Ws, l2x_Wn, l2x_b, l2h_Ws, l2h_Wn, l2h_b,
           lin_W, lin_b):
    pad = EP - E
    npt = NTILES // 2
    src = jnp.concatenate([edge_index[0].astype(jnp.int32),
                           jnp.zeros((pad,), jnp.int32)]).reshape(npt, CPB, CH)
    dst = jnp.concatenate([edge_index[1].astype(jnp.int32),
                           jnp.zeros((pad,), jnp.int32)]).reshape(npt, CPB, CH)
    wp = jnp.concatenate([edge_attr, jnp.zeros((pad,), jnp.float32)])
    wg = wp.reshape(npt, CPB * GRP, H)
    wcat = jnp.concatenate([gs_Ws, gs_Wn], axis=1)

    s0T, u0 = _k1(x, wcat)
    a1 = _seg_pass_w(_as_bf16(u0), src, dst, wg)
    emb, xa2, xab, rden = _k2(s0T, a1, jnp.tile(gs_b, T).reshape(1, D))

    a2 = _seg_pass(_as_bf16(xab), src, dst, wg)
    b1 = (l1x_b + l1h_b).reshape(1, 4 * H)
    c1, h12, h1b = _k3(xa2, a2, rden, l1x_Ws, l1x_Wn, b1)

    a3 = _seg_pass(_as_bf16(h1b), src, dst, wg)
    b2 = (l2x_b + l2h_b).reshape(1, 4 * H)
    c2, out = _k4(xa2, a2, a3, rden, c1, h12,
                  l2x_Ws, l2x_Wn, l2h_Ws, l2h_Wn, b2,
                  lin_W, lin_b.reshape(1, 1))
    return (out, c2, emb)
